# Initial kernel scaffold; baseline (speedup 1.0000x reference)
import jax
import jax.numpy as jnp
from jax.experimental import pallas as pl


def kernel(x, emb_table):
    raise NotImplementedError("write your pallas kernel here")


# speedup vs baseline: 1.5206x; 1.5206x over previous
import functools

import jax
import jax.numpy as jnp
from jax import lax
from jax.experimental import pallas as pl
from jax.experimental.pallas import tpu as pltpu
from jax.experimental.pallas import tpu_sc as plsc

MAX_LEN = 8192
HIDDEN_DIM = 1024
NUM_CORES = 2
NUM_SUBCORES = 16
NUM_WORKERS = NUM_CORES * NUM_SUBCORES
ROWS_PER_WORKER = MAX_LEN // NUM_WORKERS
CHUNK_ROWS = 32
NUM_CHUNKS = ROWS_PER_WORKER // CHUNK_ROWS


@functools.partial(
    pl.kernel,
    mesh=plsc.VectorSubcoreMesh(core_axis_name="c", subcore_axis_name="s"),
    out_type=jax.ShapeDtypeStruct((MAX_LEN, HIDDEN_DIM), jnp.float32),
    scratch_types=[
        pltpu.VMEM((CHUNK_ROWS, HIDDEN_DIM), jnp.float32),
        pltpu.VMEM((CHUNK_ROWS, HIDDEN_DIM), jnp.float32),
        pltpu.SemaphoreType.DMA,
        pltpu.SemaphoreType.DMA,
    ],
)
def _pos_emb_copy(table_hbm, out_hbm, buf0, buf1, sem0, sem1):
    wid = lax.axis_index("s") * NUM_CORES + lax.axis_index("c")
    base = wid * ROWS_PER_WORKER
    bufs = (buf0, buf1)
    sems = (sem0, sem1)
    out_descs = [None, None]
    for i in range(NUM_CHUNKS):
        b = i % 2
        if out_descs[b] is not None:
            out_descs[b].wait()
        r0 = base + i * CHUNK_ROWS
        pltpu.sync_copy(table_hbm.at[pl.ds(r0, CHUNK_ROWS)], bufs[b])
        out_descs[b] = pltpu.async_copy(
            bufs[b], out_hbm.at[pl.ds(r0, CHUNK_ROWS)], sems[b])
    for d in out_descs:
        if d is not None:
            d.wait()


def kernel(x, emb_table):
    seq_len = x.shape[1]
    out = _pos_emb_copy(emb_table)
    return out[None, :seq_len]
